# baseline (speedup 1.0000x reference)
import functools

import jax
import jax.numpy as jnp
from jax import lax
from jax.experimental import pallas as pl
from jax.experimental.pallas import tpu as pltpu
from jax.experimental.pallas import tpu_sc as plsc

_EMBED_DIM = 64
_NUM_CORES = 2
_NUM_SUBCORES = 16
_NW = _NUM_CORES * _NUM_SUBCORES
_CHUNK = 512


def _gather_kernel(n_rows):
    b_per_w = n_rows // _NW
    n_chunks = b_per_w // _CHUNK
    mesh = plsc.VectorSubcoreMesh(core_axis_name="c", subcore_axis_name="s")

    @functools.partial(
        pl.kernel,
        out_type=jax.ShapeDtypeStruct((n_rows, _EMBED_DIM), jnp.float32),
        mesh=mesh,
        scratch_types=[
            pltpu.VMEM((_CHUNK,), jnp.int32),
            pltpu.VMEM((_CHUNK, _EMBED_DIM), jnp.float32),
            pltpu.SemaphoreType.DMA,
        ],
        compiler_params=pltpu.CompilerParams(use_tc_tiling_on_sc=False),
    )
    def k(idx_hbm, table_hbm, out_hbm, idx_v, rows_v, sem):
        wid = lax.axis_index("s") * _NUM_CORES + lax.axis_index("c")
        base = wid * b_per_w

        def body(i, carry):
            off = base + i * _CHUNK
            pltpu.sync_copy(idx_hbm.at[pl.ds(off, _CHUNK)], idx_v)
            pltpu.async_copy(table_hbm.at[idx_v], rows_v, sem).wait()
            pltpu.sync_copy(rows_v, out_hbm.at[pl.ds(off, _CHUNK)])
            return carry

        lax.fori_loop(0, n_chunks, body, 0)

    return k


def kernel(text, embedding_weight):
    b, t = text.shape
    idx_flat = text.reshape(b * t).astype(jnp.int32)
    out = _gather_kernel(b * t)(idx_flat, embedding_weight)
    return out.reshape(b, t, _EMBED_DIM)


# speedup vs baseline: 1.0416x; 1.0416x over previous
import functools

import jax
import jax.numpy as jnp
from jax import lax
from jax.experimental import pallas as pl
from jax.experimental.pallas import tpu as pltpu
from jax.experimental.pallas import tpu_sc as plsc

_D = 64
_NC = 2
_NS = 16
_NW = _NC * _NS
_CHUNK = 320
_NBUF = 4
_LOOK = _NBUF - 1


def _gather_kernel(n_rows):
    b_per_w = n_rows // _NW
    n_chunks = b_per_w // _CHUNK
    n_groups = n_chunks // _NBUF
    mesh = plsc.VectorSubcoreMesh(core_axis_name="c", subcore_axis_name="s")

    @functools.partial(
        pl.kernel,
        out_type=jax.ShapeDtypeStruct((n_rows, _D), jnp.float32),
        mesh=mesh,
        scratch_types=[
            pltpu.VMEM((n_chunks, _CHUNK), jnp.int32),
            pltpu.VMEM((_NBUF, _CHUNK, _D), jnp.float32),
        ]
        + [pltpu.SemaphoreType.DMA] * (2 * _NBUF),
        compiler_params=pltpu.CompilerParams(use_tc_tiling_on_sc=False),
    )
    def k(idx_hbm, table_hbm, out_hbm, idx_v, rows, *sems):
        sem_g = sems[:_NBUF]
        sem_s = sems[_NBUF:]
        wid = lax.axis_index("s") * _NC + lax.axis_index("c")
        base = wid * b_per_w

        pltpu.sync_copy(idx_hbm.at[wid], idx_v)

        def gather(i, b):
            return pltpu.make_async_copy(
                table_hbm.at[idx_v.at[i]], rows.at[b], sem_g[b])

        def store(i, b):
            return pltpu.make_async_copy(
                rows.at[b], out_hbm.at[pl.ds(base + i * _CHUNK, _CHUNK)],
                sem_s[b])

        def step(i, b, drain, prefetch):
            gather(i, b).wait()
            store(i, b).start()
            bp = (b + _LOOK) % _NBUF
            if drain:
                store(i - 1, bp).wait()
            if prefetch:
                gather(i + _LOOK, bp).start()

        for b in range(_LOOK):
            gather(b, b).start()
        for b in range(_NBUF):
            step(b, b, drain=(b >= 1), prefetch=True)

        def group(g):
            for b in range(_NBUF):
                step(g * _NBUF + b, b, drain=True, prefetch=True)

        pl.loop(1, n_groups - 1)(group)

        i0 = (n_groups - 1) * _NBUF
        for b in range(_NBUF):
            i = i0 + b
            step(i, b, drain=(i + _LOOK < n_chunks), prefetch=(i + _LOOK < n_chunks))

        for b in range(_NBUF):
            store(i0 + b, b).wait()

    return k


def kernel(text, embedding_weight):
    b, t = text.shape
    n_rows = b * t
    b_per_w = n_rows // _NW
    n_chunks = b_per_w // _CHUNK
    idx = text.reshape(_NW, n_chunks, _CHUNK).astype(jnp.int32)
    out = _gather_kernel(n_rows)(idx, embedding_weight)
    return out.reshape(b, t, _D)


# speedup vs baseline: 1.0454x; 1.0037x over previous
import functools

import jax
import jax.numpy as jnp
from jax import lax
from jax.experimental import pallas as pl
from jax.experimental.pallas import tpu as pltpu
from jax.experimental.pallas import tpu_sc as plsc

_D = 64
_NC = 2
_NS = 16
_NW = _NC * _NS
_NBUF = 4
_LOOK = _NBUF - 1


def _gather_kernel(n_text, seq):
    rows_per_w = n_text // _NW
    mesh = plsc.VectorSubcoreMesh(core_axis_name="c", subcore_axis_name="s")

    @functools.partial(
        pl.kernel,
        out_type=jax.ShapeDtypeStruct((n_text, seq, _D), jnp.float32),
        mesh=mesh,
        scratch_types=[
            pltpu.VMEM((rows_per_w, seq), jnp.int32),
            pltpu.VMEM((_NBUF, seq, _D), jnp.float32),
        ]
        + [pltpu.SemaphoreType.DMA] * (2 * _NBUF),
        compiler_params=pltpu.CompilerParams(use_tc_tiling_on_sc=False),
    )
    def k(idx_hbm, table_hbm, out_hbm, idx_v, rows, *sems):
        sem_g = sems[:_NBUF]
        sem_s = sems[_NBUF:]
        wid = lax.axis_index("s") * _NC + lax.axis_index("c")
        base = wid * rows_per_w

        pltpu.sync_copy(idx_hbm.at[pl.ds(base, rows_per_w)], idx_v)

        def gather(i, b):
            return pltpu.make_async_copy(
                table_hbm.at[idx_v.at[i]], rows.at[b], sem_g[b])

        def store(i, b):
            return pltpu.make_async_copy(
                rows.at[b], out_hbm.at[base + i], sem_s[b])

        def step(i, b, drain, prefetch):
            gather(i, b).wait()
            store(i, b).start()
            bp = (b + _LOOK) % _NBUF
            if drain:
                store(i - 1, bp).wait()
            if prefetch:
                gather(i + _LOOK, bp).start()

        for b in range(_LOOK):
            gather(b, b).start()
        for b in range(_NBUF):
            step(b, b, drain=(b >= 1), prefetch=True)

        n_groups = rows_per_w // _NBUF

        def group(g):
            for b in range(_NBUF):
                step(g * _NBUF + b, b, drain=True, prefetch=True)

        pl.loop(1, n_groups - 1)(group)

        i0 = (n_groups - 1) * _NBUF
        for b in range(_NBUF):
            i = i0 + b
            ok = i + _LOOK < rows_per_w
            step(i, b, drain=ok, prefetch=ok)

        for b in range(_NBUF):
            store(i0 + b, b).wait()

    return k


def kernel(text, embedding_weight):
    n_text, seq = text.shape
    return _gather_kernel(n_text, seq)(text, embedding_weight)


# speedup vs baseline: 1.3853x; 1.3252x over previous
import functools

import jax
import jax.numpy as jnp
from jax import lax
from jax.experimental import pallas as pl
from jax.experimental.pallas import tpu as pltpu
from jax.experimental.pallas import tpu_sc as plsc

_D = 64
_NC = 2
_NS = 16
_NW = _NC * _NS
_NBUF = 4
_LOOK = _NBUF - 1


def _gather_kernel(n_text, seq):
    rows_per_w = n_text // _NW
    mesh = plsc.VectorSubcoreMesh(core_axis_name="c", subcore_axis_name="s")

    @functools.partial(
        pl.kernel,
        out_type=jax.ShapeDtypeStruct((n_text * seq, 2 * _D), jnp.float32),
        mesh=mesh,
        scratch_types=[
            pltpu.VMEM((rows_per_w, seq), jnp.int32),
            pltpu.VMEM((_NBUF, seq, _D), jnp.float32),
        ]
        + [pltpu.SemaphoreType.DMA] * (2 * _NBUF),
        compiler_params=pltpu.CompilerParams(use_tc_tiling_on_sc=False),
    )
    def k(idx_hbm, table_hbm, out_hbm, idx_v, rows, *sems):
        sem_g = sems[:_NBUF]
        sem_s = sems[_NBUF:]
        wid = lax.axis_index("s") * _NC + lax.axis_index("c")
        base = wid * rows_per_w

        pltpu.sync_copy(idx_hbm.at[pl.ds(base, rows_per_w)], idx_v)

        def gather(i, b):
            return pltpu.make_async_copy(
                table_hbm.at[idx_v.at[i]], rows.at[b], sem_g[b])

        def store(i, b):
            return pltpu.make_async_copy(
                rows.at[b],
                out_hbm.at[pl.ds((base + i) * seq, seq), pl.ds(0, _D)],
                sem_s[b])

        def step(i, b, drain, prefetch):
            gather(i, b).wait()
            store(i, b).start()
            bp = (b + _LOOK) % _NBUF
            if drain:
                store(i - 1, bp).wait()
            if prefetch:
                gather(i + _LOOK, bp).start()

        for b in range(_LOOK):
            gather(b, b).start()
        for b in range(_NBUF):
            step(b, b, drain=(b >= 1), prefetch=True)

        n_groups = rows_per_w // _NBUF

        def group(g):
            for b in range(_NBUF):
                step(g * _NBUF + b, b, drain=True, prefetch=True)

        pl.loop(1, n_groups - 1)(group)

        i0 = (n_groups - 1) * _NBUF
        for b in range(_NBUF):
            i = i0 + b
            ok = i + _LOOK < rows_per_w
            step(i, b, drain=ok, prefetch=ok)

        for b in range(_NBUF):
            store(i0 + b, b).wait()

    return k


def kernel(text, embedding_weight):
    n_text, seq = text.shape
    out = _gather_kernel(n_text, seq)(text, embedding_weight)
    return out.reshape(n_text, seq, 2 * _D)[:, :, :_D]
